# baseline (speedup 1.0000x reference)
import numpy as np

import jax
import jax.numpy as jnp
from jax.experimental import pallas as pl
from jax.experimental.pallas import tpu as pltpu

_TINY = np.float32(np.finfo(np.float32).tiny)
_B = 16
_T = 4096


def _threefry2x32(k0, k1, x0, x1):
    ks2 = k0 ^ k1 ^ jnp.uint32(0x1BD11BDA)
    ks = (k0, k1, ks2)
    x0 = x0 + ks[0]
    x1 = x1 + ks[1]
    rots = ((13, 15, 26, 6), (17, 29, 16, 24))
    for i in range(5):
        for r in rots[i % 2]:
            x0 = x0 + x1
            x1 = ((x1 << jnp.uint32(r)) | (x1 >> jnp.uint32(32 - r))) ^ x0
        x0 = x0 + ks[(i + 1) % 3]
        x1 = x1 + ks[(i + 2) % 3] + jnp.uint32(i + 1)
    return x0, x1


def _argmax3(a0, a1, a2):
    idx = jnp.where(a1 > a0, jnp.int32(1), jnp.int32(0))
    best = jnp.maximum(a0, a1)
    return jnp.where(a2 > best, jnp.int32(2), idx)


def _shift1(s):
    return jnp.concatenate([jnp.ones((_B, 1), jnp.int32), s[:, :-1]], axis=1)


def _compute_chain(lg_ref):
    B, T = _B, _T
    TS = T - 2

    t_lane = jax.lax.broadcasted_iota(jnp.uint32, (1, T), 1)
    zero_row = jnp.zeros((1, T), jnp.uint32)
    kb1, kb2 = _threefry2x32(jnp.uint32(0), jnp.uint32(42), zero_row, t_lane)

    r = jax.lax.broadcasted_iota(jnp.uint32, (3 * B, T), 0)
    cnt = jnp.uint32(3) * (r % jnp.uint32(B)) + r // jnp.uint32(B)
    o1, o2 = _threefry2x32(
        jnp.broadcast_to(kb1, (3 * B, T)),
        jnp.broadcast_to(kb2, (3 * B, T)),
        jnp.zeros((3 * B, T), jnp.uint32),
        cnt,
    )
    bits = o1 ^ o2

    fb = (bits >> jnp.uint32(9)) | jnp.uint32(0x3F800000)
    f = jax.lax.bitcast_convert_type(fb, jnp.float32) - jnp.float32(1.0)
    u = jnp.maximum(_TINY, f * (jnp.float32(1.0) - _TINY) + _TINY)
    g = -jnp.log(-jnp.log(u))
    g0, g1, g2 = g[0:B], g[B:2 * B], g[2 * B:3 * B]

    ld0, ld1, ld2 = lg_ref[0, 0], lg_ref[0, 1], lg_ref[0, 2]
    ls0, ls1, ls2 = lg_ref[1, 0], lg_ref[1, 1], lg_ref[1, 2]
    cd = _argmax3(g0 + ld0, g1 + ld1, g2 + ld2)
    cs = _argmax3(g0 + ls0, g1 + ls1, g2 + ls2)
    lane = jax.lax.broadcasted_iota(jnp.int32, (B, T), 1)
    valid = lane < TS
    one = jnp.int32(1)
    cd = jnp.where(valid, cd, one)
    cs = jnp.where(valid, cs, one)

    def step(s):
        sm1 = _shift1(s)
        sm2 = _shift1(sm1)
        z = (sm2 == 2) & (sm1 == 1)
        return jnp.where(z, cs, cd)

    def cond(c):
        return jnp.logical_not(c[1])

    def body(c):
        s, _ = c
        s2 = step(s)
        return s2, jnp.all(s2 == s)

    s, _ = jax.lax.while_loop(cond, body, (cd, False))

    return _shift1(s) - one


_IB = 128


def _fused_kernel(lg_ref, x_ref, y_ref, d_ref):
    b = pl.program_id(0)

    @pl.when((b == 0) & (pl.program_id(1) == 0))
    def _():
        d_ref[...] = _compute_chain(lg_ref)

    x2 = x_ref[0]
    dd = d_ref[pl.ds(b, 1), :]
    xm = pltpu.roll(x2, 1, axis=1)
    xp = pltpu.roll(x2, _T - 1, axis=1)
    y_ref[0] = jnp.where(dd < 0, xm, jnp.where(dd > 0, xp, x2))


def kernel(x, probs):
    B, I, T = x.shape

    lp = jnp.log(jnp.clip(probs, 1e-30, None))
    lg = jnp.zeros((8, 128), jnp.float32)
    lg = lg.at[0, :3].set(lp[0, 0]).at[1, :3].set(lp[2, 1])

    y = pl.pallas_call(
        _fused_kernel,
        grid=(B, I // _IB),
        in_specs=[
            pl.BlockSpec((8, 128), lambda b, i: (0, 0)),
            pl.BlockSpec((1, _IB, T), lambda b, i: (b, i, 0)),
        ],
        out_specs=pl.BlockSpec((1, _IB, T), lambda b, i: (b, i, 0)),
        out_shape=jax.ShapeDtypeStruct((B, I, T), x.dtype),
        scratch_shapes=[pltpu.VMEM((B, T), jnp.int32)],
    )(lg, x)
    return y


# speedup vs baseline: 1.1843x; 1.1843x over previous
import numpy as np

import jax
import jax.numpy as jnp
from jax.experimental import pallas as pl
from jax.experimental.pallas import tpu as pltpu

_TINY = np.float32(np.finfo(np.float32).tiny)
_B = 16
_T = 4096


def _threefry2x32(k0, k1, x0, x1):
    ks2 = k0 ^ k1 ^ jnp.uint32(0x1BD11BDA)
    ks = (k0, k1, ks2)
    x0 = x0 + ks[0]
    x1 = x1 + ks[1]
    rots = ((13, 15, 26, 6), (17, 29, 16, 24))
    for i in range(5):
        for r in rots[i % 2]:
            x0 = x0 + x1
            x1 = ((x1 << jnp.uint32(r)) | (x1 >> jnp.uint32(32 - r))) ^ x0
        x0 = x0 + ks[(i + 1) % 3]
        x1 = x1 + ks[(i + 2) % 3] + jnp.uint32(i + 1)
    return x0, x1


def _argmax3(a0, a1, a2):
    idx = jnp.where(a1 > a0, jnp.int32(1), jnp.int32(0))
    best = jnp.maximum(a0, a1)
    return jnp.where(a2 > best, jnp.int32(2), idx)


def _shift1(s):
    return jnp.concatenate([jnp.ones((_B, 1), jnp.int32), s[:, :-1]], axis=1)


def _compute_chain(lg_ref):
    B, T = _B, _T
    TS = T - 2

    t_lane = jax.lax.broadcasted_iota(jnp.uint32, (1, T), 1)
    zero_row = jnp.zeros((1, T), jnp.uint32)
    kb1, kb2 = _threefry2x32(jnp.uint32(0), jnp.uint32(42), zero_row, t_lane)

    r = jax.lax.broadcasted_iota(jnp.uint32, (3 * B, T), 0)
    cnt = jnp.uint32(3) * (r % jnp.uint32(B)) + r // jnp.uint32(B)
    o1, o2 = _threefry2x32(
        jnp.broadcast_to(kb1, (3 * B, T)),
        jnp.broadcast_to(kb2, (3 * B, T)),
        jnp.zeros((3 * B, T), jnp.uint32),
        cnt,
    )
    bits = o1 ^ o2

    fb = (bits >> jnp.uint32(9)) | jnp.uint32(0x3F800000)
    f = jax.lax.bitcast_convert_type(fb, jnp.float32) - jnp.float32(1.0)
    u = jnp.maximum(_TINY, f * (jnp.float32(1.0) - _TINY) + _TINY)
    g = -jnp.log(-jnp.log(u))
    g0, g1, g2 = g[0:B], g[B:2 * B], g[2 * B:3 * B]

    ld0, ld1, ld2 = lg_ref[0, 0], lg_ref[0, 1], lg_ref[0, 2]
    ls0, ls1, ls2 = lg_ref[1, 0], lg_ref[1, 1], lg_ref[1, 2]
    cd = _argmax3(g0 + ld0, g1 + ld1, g2 + ld2)
    cs = _argmax3(g0 + ls0, g1 + ls1, g2 + ls2)
    lane = jax.lax.broadcasted_iota(jnp.int32, (B, T), 1)
    valid = lane < TS
    one = jnp.int32(1)
    cd = jnp.where(valid, cd, one)
    cs = jnp.where(valid, cs, one)

    def step(s):
        sm1 = _shift1(s)
        sm2 = _shift1(sm1)
        z = (sm2 == 2) & (sm1 == 1)
        return jnp.where(z, cs, cd)

    def cond(c):
        return jnp.logical_not(c[1])

    def body(c):
        s, _ = c
        s2 = step(s)
        return s2, jnp.all(s2 == s)

    s, _ = jax.lax.while_loop(cond, body, (cd, False))

    return _shift1(s) - one


_BB = 2


def _fused_kernel(lg_ref, x_ref, y_ref, d_ref):
    b = pl.program_id(0)

    @pl.when(b == 0)
    def _():
        d_ref[...] = _compute_chain(lg_ref)

    for k in range(_BB):
        x2 = x_ref[k]
        dd = d_ref[pl.ds(b * _BB + k, 1), :]
        xm = pltpu.roll(x2, 1, axis=1)
        xp = pltpu.roll(x2, _T - 1, axis=1)
        y_ref[k] = jnp.where(dd < 0, xm, jnp.where(dd > 0, xp, x2))


def kernel(x, probs):
    B, I, T = x.shape

    lp = jnp.log(jnp.clip(probs, 1e-30, None))
    lg = jnp.zeros((8, 128), jnp.float32)
    lg = lg.at[0, :3].set(lp[0, 0]).at[1, :3].set(lp[2, 1])

    y = pl.pallas_call(
        _fused_kernel,
        grid=(B // _BB,),
        in_specs=[
            pl.BlockSpec((8, 128), lambda b: (0, 0)),
            pl.BlockSpec((_BB, I, T), lambda b: (b, 0, 0)),
        ],
        out_specs=pl.BlockSpec((_BB, I, T), lambda b: (b, 0, 0)),
        out_shape=jax.ShapeDtypeStruct((B, I, T), x.dtype),
        scratch_shapes=[pltpu.VMEM((B, T), jnp.int32)],
    )(lg, x)
    return y


# speedup vs baseline: 1.1857x; 1.0012x over previous
import numpy as np

import jax
import jax.numpy as jnp
from jax.experimental import pallas as pl
from jax.experimental.pallas import tpu as pltpu

_TINY = np.float32(np.finfo(np.float32).tiny)
_B = 16
_T = 4096
_BB = 2


def _threefry2x32(k0, k1, x0, x1):
    ks2 = k0 ^ k1 ^ jnp.uint32(0x1BD11BDA)
    ks = (k0, k1, ks2)
    x0 = x0 + ks[0]
    x1 = x1 + ks[1]
    rots = ((13, 15, 26, 6), (17, 29, 16, 24))
    for i in range(5):
        for r in rots[i % 2]:
            x0 = x0 + x1
            x1 = ((x1 << jnp.uint32(r)) | (x1 >> jnp.uint32(32 - r))) ^ x0
        x0 = x0 + ks[(i + 1) % 3]
        x1 = x1 + ks[(i + 2) % 3] + jnp.uint32(i + 1)
    return x0, x1


def _argmax3(a0, a1, a2):
    idx = jnp.where(a1 > a0, jnp.int32(1), jnp.int32(0))
    best = jnp.maximum(a0, a1)
    return jnp.where(a2 > best, jnp.int32(2), idx)


def _shift1(s):
    return jnp.concatenate(
        [jnp.ones((s.shape[0], 1), jnp.int32), s[:, :-1]], axis=1)


def _pair_chain(lg_ref, kb_ref, pair):
    T = _T
    TS = T - 2

    kb1 = kb_ref[0:1, :]
    kb2 = kb_ref[1:2, :]

    r = jax.lax.broadcasted_iota(jnp.uint32, (8, T), 0)
    bb = r & jnp.uint32(1)
    j = r >> jnp.uint32(1)
    gb = jnp.uint32(2) * jnp.uint32(pair) + bb
    cnt = jnp.uint32(3) * gb + j
    o1, o2 = _threefry2x32(
        jnp.broadcast_to(kb1, (8, T)),
        jnp.broadcast_to(kb2, (8, T)),
        jnp.zeros((8, T), jnp.uint32),
        cnt,
    )
    bits = o1 ^ o2

    fb = (bits >> jnp.uint32(9)) | jnp.uint32(0x3F800000)
    f = jax.lax.bitcast_convert_type(fb, jnp.float32) - jnp.float32(1.0)
    u = jnp.maximum(_TINY, f * (jnp.float32(1.0) - _TINY) + _TINY)
    g = -jnp.log(-jnp.log(u))
    g0, g1, g2 = g[0:2], g[2:4], g[4:6]

    ld0, ld1, ld2 = lg_ref[0, 0], lg_ref[0, 1], lg_ref[0, 2]
    ls0, ls1, ls2 = lg_ref[1, 0], lg_ref[1, 1], lg_ref[1, 2]
    cd = _argmax3(g0 + ld0, g1 + ld1, g2 + ld2)
    cs = _argmax3(g0 + ls0, g1 + ls1, g2 + ls2)
    lane = jax.lax.broadcasted_iota(jnp.int32, (2, T), 1)
    one = jnp.int32(1)
    cd = jnp.where(lane < TS, cd, one)
    cs = jnp.where(lane < TS, cs, one)

    def step(s):
        sm1 = _shift1(s)
        sm2 = _shift1(sm1)
        z = (sm2 == 2) & (sm1 == 1)
        return jnp.where(z, cs, cd)

    def cond(c):
        return jnp.logical_not(c[1])

    def body(c):
        s, _ = c
        s2 = step(s)
        return s2, jnp.all(s2 == s)

    s, _ = jax.lax.while_loop(cond, body, (cd, False))

    return _shift1(s) - one


def _fused_kernel(lg_ref, x_ref, y_ref, kb_ref):
    b = pl.program_id(0)

    @pl.when(b == 0)
    def _():
        t_lane = jax.lax.broadcasted_iota(jnp.uint32, (1, _T), 1)
        zero_row = jnp.zeros((1, _T), jnp.uint32)
        kb1, kb2 = _threefry2x32(
            jnp.uint32(0), jnp.uint32(42), zero_row, t_lane)
        kb_ref[0:1, :] = kb1
        kb_ref[1:2, :] = kb2

    d = _pair_chain(lg_ref, kb_ref, b)

    for k in range(_BB):
        x2 = x_ref[k]
        dd = d[k:k + 1, :]
        xm = pltpu.roll(x2, 1, axis=1)
        xp = pltpu.roll(x2, _T - 1, axis=1)
        y_ref[k] = jnp.where(dd < 0, xm, jnp.where(dd > 0, xp, x2))


def kernel(x, probs):
    B, I, T = x.shape

    lp = jnp.log(jnp.clip(probs, 1e-30, None))
    lg = jnp.zeros((8, 128), jnp.float32)
    lg = lg.at[0, :3].set(lp[0, 0]).at[1, :3].set(lp[2, 1])

    y = pl.pallas_call(
        _fused_kernel,
        grid=(B // _BB,),
        in_specs=[
            pl.BlockSpec((8, 128), lambda b: (0, 0)),
            pl.BlockSpec((_BB, I, T), lambda b: (b, 0, 0)),
        ],
        out_specs=pl.BlockSpec((_BB, I, T), lambda b: (b, 0, 0)),
        out_shape=jax.ShapeDtypeStruct((B, I, T), x.dtype),
        scratch_shapes=[pltpu.VMEM((8, T), jnp.uint32)],
    )(lg, x)
    return y
